# baseline (speedup 1.0000x reference)
import functools

import jax
import jax.numpy as jnp
from jax.experimental import pallas as pl
from jax.experimental.pallas import tpu as pltpu

_BETA = 1.0 / 9.0


def _partials_kernel(p_ref, t_ref, mask_ref, out_ref, *, beta, half_over_beta,
                     half_beta):
    diff = jnp.abs(p_ref[...] - t_ref[...])
    elem = jnp.where(diff < beta, half_over_beta * diff * diff, diff - half_beta)
    out_ref[...] = jnp.sum(elem * mask_ref[...], keepdims=True).reshape(1, 1, 1)


def _pick_block(flat_len):
    for g in (16, 32, 8, 64, 4, 128, 2, 1):
        tn = flat_len // g
        if tn * g == flat_len and tn % 5120 == 0:
            return tn, g
    return flat_len, 1


def kernel(pred, target):
    pred = pred.astype(jnp.float32)
    target = target.astype(jnp.float32)
    n = pred.shape[0]
    if n == 0:
        return jnp.float32(float("nan"))

    ncols = pred.shape[1]
    flat_len = n * ncols
    flat_p = pred.reshape(flat_len)
    flat_t = target.reshape(flat_len)
    tn, g = _pick_block(flat_len)
    if tn * g != flat_len or tn % 5120 != 0:
        tn = 5120 * max(1, -(-flat_len // (16 * 5120)))
        g = -(-flat_len // tn)
        pad = tn * g - flat_len
        flat_p = jnp.pad(flat_p, (0, pad))
        flat_t = jnp.pad(flat_t, (0, pad))

    mask = jnp.tile(
        jnp.array([1.0, 1.0, 0.0, 0.0, 0.0], dtype=jnp.float32), tn // 5)

    kernel_fn = functools.partial(
        _partials_kernel, beta=_BETA, half_over_beta=0.5 / _BETA,
        half_beta=0.5 * _BETA)
    partials = pl.pallas_call(
        kernel_fn,
        out_shape=jax.ShapeDtypeStruct((g, 1, 1), jnp.float32),
        grid=(g,),
        in_specs=[pl.BlockSpec((tn,), lambda i: (i,)),
                  pl.BlockSpec((tn,), lambda i: (i,)),
                  pl.BlockSpec((tn,), lambda i: (0,))],
        out_specs=pl.BlockSpec((1, 1, 1), lambda i: (i, 0, 0)),
        compiler_params=pltpu.CompilerParams(dimension_semantics=("parallel",)),
        cost_estimate=pl.CostEstimate(
            flops=8 * g * tn, transcendentals=0,
            bytes_accessed=8 * g * tn + 4 * tn + 4 * g),
    )(flat_p, flat_t, mask)
    return jnp.sum(partials) * (1.0 / n)


# speedup vs baseline: 32.3706x; 32.3706x over previous
import functools

import jax
import jax.numpy as jnp
from jax.experimental import pallas as pl
from jax.experimental.pallas import tpu as pltpu

_BETA = 1.0 / 9.0
_MAX_TN = 131072


def _partials_kernel(p_ref, t_ref, out_ref, *, beta, half_over_beta, half_beta,
                     n, tn):
    diff = jnp.abs(p_ref[0:2, :] - t_ref[0:2, :])
    elem = jnp.where(diff < beta, half_over_beta * diff * diff, diff - half_beta)
    cols = pl.program_id(0) * tn + jax.lax.broadcasted_iota(jnp.int32, (1, tn), 1)
    elem = jnp.where(cols < n, elem, 0.0)
    out_ref[...] = jnp.sum(elem, keepdims=True).reshape(1, 1, 1)


def _pick_tn(n):
    tn = _MAX_TN
    while tn > 128 and n % tn != 0:
        tn //= 2
    return tn if n % tn == 0 else min(n, _MAX_TN)


def kernel(pred, target):
    pred = pred.astype(jnp.float32)
    target = target.astype(jnp.float32)
    n = pred.shape[0]
    if n == 0:
        return jnp.float32(float("nan"))

    pt = pred.T
    tt = target.T
    tn = _pick_tn(n)
    g = -(-n // tn)

    kernel_fn = functools.partial(
        _partials_kernel, beta=_BETA, half_over_beta=0.5 / _BETA,
        half_beta=0.5 * _BETA, n=n, tn=tn)
    partials = pl.pallas_call(
        kernel_fn,
        out_shape=jax.ShapeDtypeStruct((g, 1, 1), jnp.float32),
        grid=(g,),
        in_specs=[pl.BlockSpec((5, tn), lambda i: (0, i)),
                  pl.BlockSpec((5, tn), lambda i: (0, i))],
        out_specs=pl.BlockSpec((1, 1, 1), lambda i: (i, 0, 0)),
        compiler_params=pltpu.CompilerParams(dimension_semantics=("parallel",)),
        cost_estimate=pl.CostEstimate(
            flops=10 * g * tn, transcendentals=0,
            bytes_accessed=16 * g * tn + 4 * g),
    )(pt, tt)
    return jnp.sum(partials) * (1.0 / n)


# speedup vs baseline: 65.7687x; 2.0317x over previous
import functools

import jax
import jax.numpy as jnp
from jax.experimental import pallas as pl
from jax.experimental.pallas import tpu as pltpu

_BETA = 1.0 / 9.0
_MAX_TN = 131072
_NCORES = 2


def _huber_sum(px, tx, *, beta, half_over_beta, half_beta):
    diff = jnp.abs(px - tx)
    elem = jnp.where(diff < beta, half_over_beta * diff * diff, diff - half_beta)
    return jnp.sum(elem, keepdims=True)


def _dma_kernel(p_hbm, t_hbm, out_ref, pbuf, tbuf, psem, tsem, *,
                beta, half_over_beta, half_beta, tn):
    i = pl.program_id(0)
    j = pl.program_id(1)
    steps = pl.num_programs(1)

    def start(step, slot):
        col0 = (i * steps + step) * tn
        pltpu.make_async_copy(
            p_hbm.at[pl.ds(0, 2), pl.ds(col0, tn)], pbuf.at[slot], psem.at[slot]
        ).start()
        pltpu.make_async_copy(
            t_hbm.at[pl.ds(0, 2), pl.ds(col0, tn)], tbuf.at[slot], tsem.at[slot]
        ).start()

    slot = jax.lax.rem(j, 2)

    @pl.when(j == 0)
    def _():
        start(j, slot)

    @pl.when(j + 1 < steps)
    def _():
        start(j + 1, 1 - slot)

    col0 = (i * steps + j) * tn
    pltpu.make_async_copy(
        p_hbm.at[pl.ds(0, 2), pl.ds(col0, tn)], pbuf.at[slot], psem.at[slot]
    ).wait()
    pltpu.make_async_copy(
        t_hbm.at[pl.ds(0, 2), pl.ds(col0, tn)], tbuf.at[slot], tsem.at[slot]
    ).wait()

    partial = _huber_sum(pbuf[slot], tbuf[slot], beta=beta,
                         half_over_beta=half_over_beta, half_beta=half_beta)

    @pl.when(j == 0)
    def _():
        out_ref[...] = jnp.zeros_like(out_ref)

    out_ref[...] += partial.reshape(1, 1, 1)


def _blocked_kernel(p_ref, t_ref, out_ref, *, beta, half_over_beta, half_beta,
                    n, tn):
    diff = jnp.abs(p_ref[0:2, :] - t_ref[0:2, :])
    elem = jnp.where(diff < beta, half_over_beta * diff * diff, diff - half_beta)
    cols = pl.program_id(0) * tn + jax.lax.broadcasted_iota(jnp.int32, (1, tn), 1)
    elem = jnp.where(cols < n, elem, 0.0)
    out_ref[...] = jnp.sum(elem, keepdims=True).reshape(1, 1, 1)


def _pick_tn(n, multiple_of):
    tn = _MAX_TN
    while tn > 128 and n % (tn * multiple_of) != 0:
        tn //= 2
    return tn


def kernel(pred, target):
    pred = pred.astype(jnp.float32)
    target = target.astype(jnp.float32)
    n = pred.shape[0]
    if n == 0:
        return jnp.float32(float("nan"))

    pt = pred.T
    tt = target.T
    consts = dict(beta=_BETA, half_over_beta=0.5 / _BETA, half_beta=0.5 * _BETA)

    tn = _pick_tn(n, _NCORES)
    if n % (tn * _NCORES) == 0:
        steps = n // (tn * _NCORES)
        kernel_fn = functools.partial(_dma_kernel, tn=tn, **consts)
        partials = pl.pallas_call(
            kernel_fn,
            out_shape=jax.ShapeDtypeStruct((_NCORES, 1, 1), jnp.float32),
            grid=(_NCORES, steps),
            in_specs=[pl.BlockSpec(memory_space=pltpu.MemorySpace.HBM),
                      pl.BlockSpec(memory_space=pltpu.MemorySpace.HBM)],
            out_specs=pl.BlockSpec((1, 1, 1), lambda i, j: (i, 0, 0)),
            scratch_shapes=[
                pltpu.VMEM((2, 2, tn), jnp.float32),
                pltpu.VMEM((2, 2, tn), jnp.float32),
                pltpu.SemaphoreType.DMA((2,)),
                pltpu.SemaphoreType.DMA((2,)),
            ],
            compiler_params=pltpu.CompilerParams(
                dimension_semantics=("parallel", "arbitrary")),
            cost_estimate=pl.CostEstimate(
                flops=10 * n * 2, transcendentals=0,
                bytes_accessed=16 * n + 4 * _NCORES),
        )(pt, tt)
        return jnp.sum(partials) * (1.0 / n)

    tn = _pick_tn(n, 1)
    g = -(-n // tn)
    kernel_fn = functools.partial(_blocked_kernel, n=n, tn=tn, **consts)
    partials = pl.pallas_call(
        kernel_fn,
        out_shape=jax.ShapeDtypeStruct((g, 1, 1), jnp.float32),
        grid=(g,),
        in_specs=[pl.BlockSpec((5, tn), lambda i: (0, i)),
                  pl.BlockSpec((5, tn), lambda i: (0, i))],
        out_specs=pl.BlockSpec((1, 1, 1), lambda i: (i, 0, 0)),
        compiler_params=pltpu.CompilerParams(dimension_semantics=("parallel",)),
        cost_estimate=pl.CostEstimate(
            flops=10 * g * tn, transcendentals=0,
            bytes_accessed=16 * g * tn + 4 * g),
    )(pt, tt)
    return jnp.sum(partials) * (1.0 / n)


# speedup vs baseline: 71.1098x; 1.0812x over previous
import functools

import jax
import jax.numpy as jnp
from jax.experimental import pallas as pl
from jax.experimental.pallas import tpu as pltpu

_BETA = 1.0 / 9.0
_MAX_TN = 262144
_NCORES = 2


def _huber_sum(px, tx, *, beta, half_over_beta, half_beta):
    diff = jnp.abs(px - tx)
    elem = jnp.where(diff < beta, half_over_beta * diff * diff, diff - half_beta)
    return jnp.sum(elem, keepdims=True)


def _dma_kernel(p_hbm, t_hbm, out_ref, pbuf, tbuf, psem, tsem, *,
                beta, half_over_beta, half_beta, tn):
    i = pl.program_id(0)
    j = pl.program_id(1)
    steps = pl.num_programs(1)

    def start(step, slot):
        col0 = (i * steps + step) * tn
        pltpu.make_async_copy(
            p_hbm.at[pl.ds(0, 2), pl.ds(col0, tn)], pbuf.at[slot], psem.at[slot]
        ).start()
        pltpu.make_async_copy(
            t_hbm.at[pl.ds(0, 2), pl.ds(col0, tn)], tbuf.at[slot], tsem.at[slot]
        ).start()

    slot = jax.lax.rem(j, 2)

    @pl.when(j == 0)
    def _():
        start(j, slot)

    @pl.when(j + 1 < steps)
    def _():
        start(j + 1, 1 - slot)

    col0 = (i * steps + j) * tn
    pltpu.make_async_copy(
        p_hbm.at[pl.ds(0, 2), pl.ds(col0, tn)], pbuf.at[slot], psem.at[slot]
    ).wait()
    pltpu.make_async_copy(
        t_hbm.at[pl.ds(0, 2), pl.ds(col0, tn)], tbuf.at[slot], tsem.at[slot]
    ).wait()

    partial = _huber_sum(pbuf[slot], tbuf[slot], beta=beta,
                         half_over_beta=half_over_beta, half_beta=half_beta)

    @pl.when(j == 0)
    def _():
        out_ref[...] = jnp.zeros_like(out_ref)

    out_ref[...] += partial.reshape(1, 1, 1)


def _blocked_kernel(p_ref, t_ref, out_ref, *, beta, half_over_beta, half_beta,
                    n, tn):
    diff = jnp.abs(p_ref[0:2, :] - t_ref[0:2, :])
    elem = jnp.where(diff < beta, half_over_beta * diff * diff, diff - half_beta)
    cols = pl.program_id(0) * tn + jax.lax.broadcasted_iota(jnp.int32, (1, tn), 1)
    elem = jnp.where(cols < n, elem, 0.0)
    out_ref[...] = jnp.sum(elem, keepdims=True).reshape(1, 1, 1)


def _pick_tn(n, multiple_of):
    tn = _MAX_TN
    while tn > 128 and n % (tn * multiple_of) != 0:
        tn //= 2
    return tn


def kernel(pred, target):
    pred = pred.astype(jnp.float32)
    target = target.astype(jnp.float32)
    n = pred.shape[0]
    if n == 0:
        return jnp.float32(float("nan"))

    pt = pred.T
    tt = target.T
    consts = dict(beta=_BETA, half_over_beta=0.5 / _BETA, half_beta=0.5 * _BETA)

    tn = _pick_tn(n, _NCORES)
    if n % (tn * _NCORES) == 0:
        steps = n // (tn * _NCORES)
        kernel_fn = functools.partial(_dma_kernel, tn=tn, **consts)
        partials = pl.pallas_call(
            kernel_fn,
            out_shape=jax.ShapeDtypeStruct((_NCORES, 1, 1), jnp.float32),
            grid=(_NCORES, steps),
            in_specs=[pl.BlockSpec(memory_space=pltpu.MemorySpace.HBM),
                      pl.BlockSpec(memory_space=pltpu.MemorySpace.HBM)],
            out_specs=pl.BlockSpec((1, 1, 1), lambda i, j: (i, 0, 0)),
            scratch_shapes=[
                pltpu.VMEM((2, 2, tn), jnp.float32),
                pltpu.VMEM((2, 2, tn), jnp.float32),
                pltpu.SemaphoreType.DMA((2,)),
                pltpu.SemaphoreType.DMA((2,)),
            ],
            compiler_params=pltpu.CompilerParams(
                dimension_semantics=("parallel", "arbitrary")),
            cost_estimate=pl.CostEstimate(
                flops=10 * n * 2, transcendentals=0,
                bytes_accessed=16 * n + 4 * _NCORES),
        )(pt, tt)
        return jnp.sum(partials) * (1.0 / n)

    tn = _pick_tn(n, 1)
    g = -(-n // tn)
    kernel_fn = functools.partial(_blocked_kernel, n=n, tn=tn, **consts)
    partials = pl.pallas_call(
        kernel_fn,
        out_shape=jax.ShapeDtypeStruct((g, 1, 1), jnp.float32),
        grid=(g,),
        in_specs=[pl.BlockSpec((5, tn), lambda i: (0, i)),
                  pl.BlockSpec((5, tn), lambda i: (0, i))],
        out_specs=pl.BlockSpec((1, 1, 1), lambda i: (i, 0, 0)),
        compiler_params=pltpu.CompilerParams(dimension_semantics=("parallel",)),
        cost_estimate=pl.CostEstimate(
            flops=10 * g * tn, transcendentals=0,
            bytes_accessed=16 * g * tn + 4 * g),
    )(pt, tt)
    return jnp.sum(partials) * (1.0 / n)
